# baseline (speedup 1.0000x reference)
import functools

import jax
import jax.numpy as jnp
from jax import lax
from jax.experimental import pallas as pl
from jax.experimental.pallas import tpu as pltpu
from jax.experimental.pallas import tpu_sc as plsc

_EPS = 1e-7
_B = 16
_N = 22743
_C = 85
_CHUNK = 2048
_OUT_ROWS = 1424
_QSTART = (0, 5632, 11392, 17024)
_QEND = (5632, 11392, 17024, _N)


def _emit_range(s, slot, astart, aend, x_ref, t_ref, o_ref, rbufs, bx, bt,
                sgx, sgt, ssx, sst):
    alen = aend - astart
    schedule = []
    nk = (alen + _CHUNK - 1) // _CHUNK
    for k in range(nk):
        clen = min(_CHUNK, alen - k * _CHUNK)
        nm = (clen + 127) // 128
        for m in range(nm):
            schedule.append((k, m, min(128, clen - 128 * m), m == nm - 1,
                             (clen + 15) // 16))

    def issue(i):
        k, m, slen, _, _ = schedule[i]
        rx, rt = rbufs[i % 2]
        g_x = pltpu.make_async_copy(
            x_ref.at[s, pl.ds(astart + _CHUNK * k + 128 * m, slen), :],
            rx.at[pl.ds(0, slen), :], sgx)
        g_t = pltpu.make_async_copy(
            t_ref.at[s, pl.ds(astart + _CHUNK * k + 128 * m, slen), :],
            rt.at[pl.ds(0, slen), :], sgt)
        g_x.start()
        g_t.start()
        return g_x, g_t

    wid = s - 8
    pending = issue(0)
    out_pending = []
    for i, (k, m, slen, last, crows) in enumerate(schedule):
        g_x, g_t = pending
        if i + 1 < len(schedule):
            pending = issue(i + 1)
        g_x.wait()
        g_t.wait()
        if m == 0:
            for cp in out_pending:
                cp.wait()
            out_pending = []
        rx, rt = rbufs[i % 2]
        pltpu.sync_copy(rx.at[pl.ds(0, slen), pl.ds(0, 8)],
                        bx.at[slot, pl.ds(0, slen), pl.ds(8 * m, 8)])
        pltpu.sync_copy(rt.at[pl.ds(0, slen), pl.ds(0, 8)],
                        bt.at[slot, pl.ds(0, slen), pl.ds(8 * m, 8)])
        if last:
            out_row = astart // 16 + k * 128
            nrows = (crows + 7) // 8 * 8
            sc_x = pltpu.make_async_copy(
                bx.at[slot, pl.ds(0, nrows), :],
                o_ref.at[0, wid, pl.ds(out_row, nrows), :], ssx)
            sc_t = pltpu.make_async_copy(
                bt.at[slot, pl.ds(0, nrows), :],
                o_ref.at[1, wid, pl.ds(out_row, nrows), :], sst)
            sc_x.start()
            sc_t.start()
            out_pending = [sc_x, sc_t]
    for cp in out_pending:
        cp.wait()


@functools.partial(
    pl.kernel,
    mesh=plsc.VectorSubcoreMesh(core_axis_name="c", subcore_axis_name="s"),
    out_type=jax.ShapeDtypeStruct((2, 8, _OUT_ROWS, 128), jnp.float32),
    scratch_types=[
        pltpu.VMEM((128, _C), jnp.float32),
        pltpu.VMEM((128, _C), jnp.float32),
        pltpu.VMEM((128, _C), jnp.float32),
        pltpu.VMEM((128, _C), jnp.float32),
        pltpu.VMEM_SHARED((16, 128, 128), jnp.float32),
        pltpu.VMEM_SHARED((16, 128, 128), jnp.float32),
        pltpu.SemaphoreType.DMA,
        pltpu.SemaphoreType.DMA,
        pltpu.SemaphoreType.DMA,
        pltpu.SemaphoreType.DMA,
    ],
)
def _sc_compact(x_hbm, t_hbm, o_hbm, rx0, rt0, rx1, rt1, bx, bt,
                sgx, sgt, ssx, sst):
    c = lax.axis_index("c")
    s = lax.axis_index("s")
    rbufs = [(rx0, rt0), (rx1, rt1)]

    for qbase, smin in ((0, 0), (2, 8)):
        for ci in range(2):
            q = qbase + ci

            @pl.when((c == ci) & (s >= smin) & (s < smin + 8))
            def _go(q=q, smin=smin):
                batch = s + (8 - smin)
                _emit_range(batch, s, _QSTART[q], _QEND[q], x_hbm, t_hbm,
                            o_hbm, rbufs, bx, bt, sgx, sgt, ssx, sst)


def _bce_partials(xp, tp, ok, mb_sel):
    pq = jnp.clip(jnp.where(ok, xp, 0.5), _EPS, 1.0 - _EPS)
    tq = jnp.where(ok, tp, 0.5)
    bce = -(tq * jnp.log(pq) + (1.0 - tq) * jnp.log(1.0 - pq))
    b = mb_sel * (tq > 0.0).astype(jnp.float32)
    mb = (jnp.roll(b, -1, axis=1) + jnp.roll(b, -2, axis=1)
          + jnp.roll(b, -3, axis=1) + jnp.roll(b, -4, axis=1))
    return jnp.sum(bce * mb), jnp.sum(b)


def _tc_direct_body(x_ref, t_ref, out_ref, acc_ref):
    bi = pl.program_id(0)
    kb = pl.program_id(1)

    @pl.when((bi == 0) & (kb == 0))
    def _init():
        acc_ref[0] = 0.0
        acc_ref[1] = 0.0

    xb = x_ref[0]
    tb = t_ref[0]
    xp = jnp.concatenate(
        [xb[128 * j:128 * (j + 1), 0:8] for j in range(16)], axis=1)
    tp = jnp.concatenate(
        [tb[128 * j:128 * (j + 1), 0:8] for j in range(16)], axis=1)

    lane = jax.lax.broadcasted_iota(jnp.int32, (128, 128), 1)
    q = jax.lax.broadcasted_iota(jnp.int32, (128, 128), 0)
    a = kb * _CHUNK + (lane // 8) * 128 + q
    valid = a < _N
    ok = valid
    mb_sel = (valid & (lane % 8 == 4)).astype(jnp.float32)

    ds, db = _bce_partials(xp, tp, ok, mb_sel)
    acc_ref[0] += ds
    acc_ref[1] += db

    @pl.when((bi == pl.num_programs(0) - 1) & (kb == pl.num_programs(1) - 1))
    def _fin():
        out_ref[0] = acc_ref[0]
        out_ref[1] = acc_ref[1]


def _tc_direct(x, target):
    nkb = (_N + _CHUNK - 1) // _CHUNK
    return pl.pallas_call(
        _tc_direct_body,
        grid=(8, nkb),
        in_specs=[
            pl.BlockSpec((1, _CHUNK, _C), lambda b, k: (b, k, 0)),
            pl.BlockSpec((1, _CHUNK, _C), lambda b, k: (b, k, 0)),
        ],
        out_specs=pl.BlockSpec(memory_space=pltpu.SMEM),
        out_shape=jax.ShapeDtypeStruct((2,), jnp.float32),
        scratch_shapes=[pltpu.SMEM((2,), jnp.float32)],
        compiler_params=pltpu.CompilerParams(
            dimension_semantics=("arbitrary", "arbitrary"),
        ),
    )(x, target)


def _tc_packed_body(xt_ref, part_ref, out_ref, acc_ref, mv_ref, mb_ref):
    i = pl.program_id(0)

    @pl.when(i == 0)
    def _init():
        acc_ref[0] = part_ref[0]
        acc_ref[1] = part_ref[1]
        row = jax.lax.broadcasted_iota(jnp.int32, (_OUT_ROWS, 128), 0)
        lane = jax.lax.broadcasted_iota(jnp.int32, (_OUT_ROWS, 128), 1)
        astart = jnp.zeros_like(row)
        base = jnp.zeros_like(row)
        aend = jnp.full_like(row, _QEND[0])
        for qi in range(1, 4):
            sel = row >= _QSTART[qi] // 16
            astart = jnp.where(sel, _QSTART[qi], astart)
            base = jnp.where(sel, _QSTART[qi] // 16, base)
            aend = jnp.where(sel, _QEND[qi], aend)
        rr = row - base
        a = astart + (rr // 128) * _CHUNK + (lane // 8) * 128 + (rr % 128)
        valid = a < aend
        mv_ref[...] = valid.astype(jnp.float32)
        mb_ref[...] = (valid & (lane % 8 == 4)).astype(jnp.float32)

    ok = mv_ref[...] > 0.0
    ds, db = _bce_partials(xt_ref[0, 0], xt_ref[1, 0], ok, mb_ref[...])
    acc_ref[0] += ds
    acc_ref[1] += db

    @pl.when(i == pl.num_programs(0) - 1)
    def _fin():
        out_ref[0, 0] = acc_ref[0] / jnp.maximum(acc_ref[1] * 2.0, 1.0)


def _tc_packed(packed, partials):
    return pl.pallas_call(
        _tc_packed_body,
        grid=(8,),
        in_specs=[
            pl.BlockSpec((2, 1, _OUT_ROWS, 128), lambda i: (0, i, 0, 0)),
            pl.BlockSpec(memory_space=pltpu.SMEM),
        ],
        out_specs=pl.BlockSpec(memory_space=pltpu.SMEM),
        out_shape=jax.ShapeDtypeStruct((1, 1), jnp.float32),
        scratch_shapes=[
            pltpu.SMEM((2,), jnp.float32),
            pltpu.VMEM((_OUT_ROWS, 128), jnp.float32),
            pltpu.VMEM((_OUT_ROWS, 128), jnp.float32),
        ],
        compiler_params=pltpu.CompilerParams(
            dimension_semantics=("arbitrary",),
        ),
    )(packed, partials)


def _tc_all_body(x0_ref, t0_ref, x1_ref, t1_ref, out_ref, acc_ref):
    bi = pl.program_id(0)
    kb = pl.program_id(1)

    @pl.when((bi == 0) & (kb == 0))
    def _init():
        acc_ref[0] = 0.0
        acc_ref[1] = 0.0

    lane = jax.lax.broadcasted_iota(jnp.int32, (128, 128), 1)
    q = jax.lax.broadcasted_iota(jnp.int32, (128, 128), 0)

    for half, (x_ref, t_ref) in enumerate(((x0_ref, t0_ref),
                                           (x1_ref, t1_ref))):
        xb = x_ref[0]
        tb = t_ref[0]
        xp = jnp.concatenate(
            [xb[128 * j:128 * (j + 1), 0:8] for j in range(16)], axis=1)
        tp = jnp.concatenate(
            [tb[128 * j:128 * (j + 1), 0:8] for j in range(16)], axis=1)
        a = (2 * kb + half) * _CHUNK + (lane // 8) * 128 + q
        valid = a < _N
        mb_sel = (valid & (lane % 8 == 4)).astype(jnp.float32)
        ds, db = _bce_partials(xp, tp, valid, mb_sel)
        acc_ref[0] += ds
        acc_ref[1] += db

    @pl.when((bi == pl.num_programs(0) - 1) & (kb == pl.num_programs(1) - 1))
    def _fin():
        out_ref[0, 0] = acc_ref[0] / jnp.maximum(acc_ref[1] * 2.0, 1.0)


def _tc_all(x, target):
    nkb = (_N + 2 * _CHUNK - 1) // (2 * _CHUNK)
    return pl.pallas_call(
        _tc_all_body,
        grid=(_B, nkb),
        in_specs=[
            pl.BlockSpec((1, _CHUNK, _C), lambda b, k: (b, 2 * k, 0)),
            pl.BlockSpec((1, _CHUNK, _C), lambda b, k: (b, 2 * k, 0)),
            pl.BlockSpec((1, _CHUNK, _C), lambda b, k: (b, 2 * k + 1, 0)),
            pl.BlockSpec((1, _CHUNK, _C), lambda b, k: (b, 2 * k + 1, 0)),
        ],
        out_specs=pl.BlockSpec(memory_space=pltpu.SMEM),
        out_shape=jax.ShapeDtypeStruct((1, 1), jnp.float32),
        scratch_shapes=[pltpu.SMEM((2,), jnp.float32)],
        compiler_params=pltpu.CompilerParams(
            dimension_semantics=("arbitrary", "arbitrary"),
        ),
    )(x, target, x, target)


def kernel(x, target):
    return _tc_all(x, target)[0, 0]


# speedup vs baseline: 1.0299x; 1.0299x over previous
import functools

import jax
import jax.numpy as jnp
from jax import lax
from jax.experimental import pallas as pl
from jax.experimental.pallas import tpu as pltpu
from jax.experimental.pallas import tpu_sc as plsc

_EPS = 1e-7
_B = 16
_N = 22743
_C = 85
_CHUNK = 2048
_OUT_ROWS = 1424
_QSTART = (0, 5632, 11392, 17024)
_QEND = (5632, 11392, 17024, _N)


def _emit_range(s, slot, astart, aend, x_ref, t_ref, o_ref, rbufs, bx, bt,
                sgx, sgt, ssx, sst):
    alen = aend - astart
    schedule = []
    nk = (alen + _CHUNK - 1) // _CHUNK
    for k in range(nk):
        clen = min(_CHUNK, alen - k * _CHUNK)
        nm = (clen + 127) // 128
        for m in range(nm):
            schedule.append((k, m, min(128, clen - 128 * m), m == nm - 1,
                             (clen + 15) // 16))

    def issue(i):
        k, m, slen, _, _ = schedule[i]
        rx, rt = rbufs[i % 2]
        g_x = pltpu.make_async_copy(
            x_ref.at[s, pl.ds(astart + _CHUNK * k + 128 * m, slen), :],
            rx.at[pl.ds(0, slen), :], sgx)
        g_t = pltpu.make_async_copy(
            t_ref.at[s, pl.ds(astart + _CHUNK * k + 128 * m, slen), :],
            rt.at[pl.ds(0, slen), :], sgt)
        g_x.start()
        g_t.start()
        return g_x, g_t

    wid = s - 8
    pending = issue(0)
    out_pending = []
    for i, (k, m, slen, last, crows) in enumerate(schedule):
        g_x, g_t = pending
        if i + 1 < len(schedule):
            pending = issue(i + 1)
        g_x.wait()
        g_t.wait()
        if m == 0:
            for cp in out_pending:
                cp.wait()
            out_pending = []
        rx, rt = rbufs[i % 2]
        pltpu.sync_copy(rx.at[pl.ds(0, slen), pl.ds(0, 8)],
                        bx.at[slot, pl.ds(0, slen), pl.ds(8 * m, 8)])
        pltpu.sync_copy(rt.at[pl.ds(0, slen), pl.ds(0, 8)],
                        bt.at[slot, pl.ds(0, slen), pl.ds(8 * m, 8)])
        if last:
            out_row = astart // 16 + k * 128
            nrows = (crows + 7) // 8 * 8
            sc_x = pltpu.make_async_copy(
                bx.at[slot, pl.ds(0, nrows), :],
                o_ref.at[0, wid, pl.ds(out_row, nrows), :], ssx)
            sc_t = pltpu.make_async_copy(
                bt.at[slot, pl.ds(0, nrows), :],
                o_ref.at[1, wid, pl.ds(out_row, nrows), :], sst)
            sc_x.start()
            sc_t.start()
            out_pending = [sc_x, sc_t]
    for cp in out_pending:
        cp.wait()


@functools.partial(
    pl.kernel,
    mesh=plsc.VectorSubcoreMesh(core_axis_name="c", subcore_axis_name="s"),
    out_type=jax.ShapeDtypeStruct((2, 8, _OUT_ROWS, 128), jnp.float32),
    scratch_types=[
        pltpu.VMEM((128, _C), jnp.float32),
        pltpu.VMEM((128, _C), jnp.float32),
        pltpu.VMEM((128, _C), jnp.float32),
        pltpu.VMEM((128, _C), jnp.float32),
        pltpu.VMEM_SHARED((16, 128, 128), jnp.float32),
        pltpu.VMEM_SHARED((16, 128, 128), jnp.float32),
        pltpu.SemaphoreType.DMA,
        pltpu.SemaphoreType.DMA,
        pltpu.SemaphoreType.DMA,
        pltpu.SemaphoreType.DMA,
    ],
)
def _sc_compact(x_hbm, t_hbm, o_hbm, rx0, rt0, rx1, rt1, bx, bt,
                sgx, sgt, ssx, sst):
    c = lax.axis_index("c")
    s = lax.axis_index("s")
    rbufs = [(rx0, rt0), (rx1, rt1)]

    for qbase, smin in ((0, 0), (2, 8)):
        for ci in range(2):
            q = qbase + ci

            @pl.when((c == ci) & (s >= smin) & (s < smin + 8))
            def _go(q=q, smin=smin):
                batch = s + (8 - smin)
                _emit_range(batch, s, _QSTART[q], _QEND[q], x_hbm, t_hbm,
                            o_hbm, rbufs, bx, bt, sgx, sgt, ssx, sst)


def _bce_partials(xp, tp, ok, mb_sel):
    pq = jnp.clip(jnp.where(ok, xp, 0.5), _EPS, 1.0 - _EPS)
    tq = jnp.where(ok, tp, 0.5)
    bce = -(tq * jnp.log(pq) + (1.0 - tq) * jnp.log(1.0 - pq))
    b = mb_sel * (tq > 0.0).astype(jnp.float32)
    mb = (jnp.roll(b, -1, axis=1) + jnp.roll(b, -2, axis=1)
          + jnp.roll(b, -3, axis=1) + jnp.roll(b, -4, axis=1))
    return jnp.sum(bce * mb), jnp.sum(b)


def _tc_direct_body(x_ref, t_ref, out_ref, acc_ref):
    bi = pl.program_id(0)
    kb = pl.program_id(1)

    @pl.when((bi == 0) & (kb == 0))
    def _init():
        acc_ref[0] = 0.0
        acc_ref[1] = 0.0

    xb = x_ref[0]
    tb = t_ref[0]
    xp = jnp.concatenate(
        [xb[128 * j:128 * (j + 1), 0:8] for j in range(16)], axis=1)
    tp = jnp.concatenate(
        [tb[128 * j:128 * (j + 1), 0:8] for j in range(16)], axis=1)

    lane = jax.lax.broadcasted_iota(jnp.int32, (128, 128), 1)
    q = jax.lax.broadcasted_iota(jnp.int32, (128, 128), 0)
    a = kb * _CHUNK + (lane // 8) * 128 + q
    valid = a < _N
    ok = valid
    mb_sel = (valid & (lane % 8 == 4)).astype(jnp.float32)

    ds, db = _bce_partials(xp, tp, ok, mb_sel)
    acc_ref[0] += ds
    acc_ref[1] += db

    @pl.when((bi == pl.num_programs(0) - 1) & (kb == pl.num_programs(1) - 1))
    def _fin():
        out_ref[0] = acc_ref[0]
        out_ref[1] = acc_ref[1]


def _tc_direct(x, target):
    nkb = (_N + _CHUNK - 1) // _CHUNK
    return pl.pallas_call(
        _tc_direct_body,
        grid=(8, nkb),
        in_specs=[
            pl.BlockSpec((1, _CHUNK, _C), lambda b, k: (b, k, 0)),
            pl.BlockSpec((1, _CHUNK, _C), lambda b, k: (b, k, 0)),
        ],
        out_specs=pl.BlockSpec(memory_space=pltpu.SMEM),
        out_shape=jax.ShapeDtypeStruct((2,), jnp.float32),
        scratch_shapes=[pltpu.SMEM((2,), jnp.float32)],
        compiler_params=pltpu.CompilerParams(
            dimension_semantics=("arbitrary", "arbitrary"),
        ),
    )(x, target)


def _tc_packed_body(xt_ref, part_ref, out_ref, acc_ref, mv_ref, mb_ref):
    i = pl.program_id(0)

    @pl.when(i == 0)
    def _init():
        acc_ref[0] = part_ref[0]
        acc_ref[1] = part_ref[1]
        row = jax.lax.broadcasted_iota(jnp.int32, (_OUT_ROWS, 128), 0)
        lane = jax.lax.broadcasted_iota(jnp.int32, (_OUT_ROWS, 128), 1)
        astart = jnp.zeros_like(row)
        base = jnp.zeros_like(row)
        aend = jnp.full_like(row, _QEND[0])
        for qi in range(1, 4):
            sel = row >= _QSTART[qi] // 16
            astart = jnp.where(sel, _QSTART[qi], astart)
            base = jnp.where(sel, _QSTART[qi] // 16, base)
            aend = jnp.where(sel, _QEND[qi], aend)
        rr = row - base
        a = astart + (rr // 128) * _CHUNK + (lane // 8) * 128 + (rr % 128)
        valid = a < aend
        mv_ref[...] = valid.astype(jnp.float32)
        mb_ref[...] = (valid & (lane % 8 == 4)).astype(jnp.float32)

    ok = mv_ref[...] > 0.0
    ds, db = _bce_partials(xt_ref[0, 0], xt_ref[1, 0], ok, mb_ref[...])
    acc_ref[0] += ds
    acc_ref[1] += db

    @pl.when(i == pl.num_programs(0) - 1)
    def _fin():
        out_ref[0, 0] = acc_ref[0] / jnp.maximum(acc_ref[1] * 2.0, 1.0)


def _tc_packed(packed, partials):
    return pl.pallas_call(
        _tc_packed_body,
        grid=(8,),
        in_specs=[
            pl.BlockSpec((2, 1, _OUT_ROWS, 128), lambda i: (0, i, 0, 0)),
            pl.BlockSpec(memory_space=pltpu.SMEM),
        ],
        out_specs=pl.BlockSpec(memory_space=pltpu.SMEM),
        out_shape=jax.ShapeDtypeStruct((1, 1), jnp.float32),
        scratch_shapes=[
            pltpu.SMEM((2,), jnp.float32),
            pltpu.VMEM((_OUT_ROWS, 128), jnp.float32),
            pltpu.VMEM((_OUT_ROWS, 128), jnp.float32),
        ],
        compiler_params=pltpu.CompilerParams(
            dimension_semantics=("arbitrary",),
        ),
    )(packed, partials)


def kernel(x, target):
    partials = _tc_direct(x, target)
    packed = _sc_compact(x, target)
    return _tc_packed(packed, partials)[0, 0]
